# Initial kernel scaffold; baseline (speedup 1.0000x reference)
import jax
import jax.numpy as jnp
from jax.experimental import pallas as pl


def kernel(queries, keys, values, Wq, bq, Wk, bk, Wv, bv, Wo, bo, index_sample):
    raise NotImplementedError("write your pallas kernel here")


# speedup vs baseline: 1.5983x; 1.5983x over previous
import math

import jax
import jax.numpy as jnp
from jax.experimental import pallas as pl
from jax.experimental.pallas import tpu as pltpu

_HIGHEST = jax.lax.Precision.DEFAULT
_F32 = jnp.float32


def _dot(a, b, dims):
    return jax.lax.dot_general(a, b, (dims, ((), ())), precision=_HIGHEST,
                               preferred_element_type=_F32)


def _make_proj(H, DK, TM, DM):
    def body(x_ref, w_ref, b_ref, o_ref):
        y = _dot(x_ref[0], w_ref[...], ((1,), (1,))) + b_ref[...]
        o_ref[0] = jnp.transpose(y.reshape(TM, H, DK), (1, 0, 2))
    return body


def _proj(x3d, W, b2d, H, DK, tm):
    B, L, DM = x3d.shape
    return pl.pallas_call(
        _make_proj(H, DK, tm, DM),
        grid=(B, L // tm),
        in_specs=[
            pl.BlockSpec((1, tm, DM), lambda b, i: (b, i, 0)),
            pl.BlockSpec((DM, DM), lambda b, i: (0, 0)),
            pl.BlockSpec((1, DM), lambda b, i: (0, 0)),
        ],
        out_specs=pl.BlockSpec((1, H, tm, DK), lambda b, i: (b, 0, i, 0)),
        out_shape=jax.ShapeDtypeStruct((B, H, L, DK), _F32),
    )(x3d, W, b2d)


def _make_gather_ks(H, SK, SKP):
    def body(k_ref, idx_ref, o_ref):
        rows = [k_ref[0, :, pl.ds(idx_ref[0, i], 1), :] for i in range(SKP)]
        o_ref[0] = jnp.concatenate(rows, axis=1)
    return body


def _make_score(H, DK, SK, TL):
    def body(q_ref, ks_ref, m_ref):
        rows = []
        for h in range(H):
            s = _dot(q_ref[0, h], ks_ref[0, h, :SK, :], ((1,), (1,)))
            rows.append(jnp.max(s, axis=1) - jnp.sum(s, axis=1) * (1.0 / SK))
        m_ref[0] = jnp.stack(rows, axis=0)
    return body


def _make_topk(H, L, NT, NTP):
    def body(m_ref, o_ref):
        m = m_ref[0]
        iota = jax.lax.broadcasted_iota(jnp.int32, (H, L), 1)
        lane = jax.lax.broadcasted_iota(jnp.int32, (H, NTP), 1)
        acc = jnp.zeros((H, NTP), jnp.int32)
        for t in range(NT):
            mx = jnp.max(m, axis=1, keepdims=True)
            idx = jnp.min(jnp.where(m >= mx, iota, L), axis=1)
            acc = jnp.where(lane == t, idx[:, None], acc)
            m = jnp.where(iota == idx[:, None], -3.4e38, m)
        o_ref[0] = acc
    return body


def _make_attn(L, DM, DK, NT, scale):
    def body(q_ref, k_ref, v_ref, idx_ref, wo_ref, bo_ref, o_ref):
        b = pl.program_id(0)
        h = pl.program_id(1)

        @pl.when(h == 0)
        def _():
            o_ref[...] = jnp.broadcast_to(bo_ref[...][None], (1, L, DM))

        ids = [idx_ref[b, h, i] for i in range(NT)]
        qr = jnp.concatenate(
            [q_ref[0, 0, pl.ds(ids[i], 1), :] for i in range(NT)], axis=0)
        logits = _dot(qr, k_ref[0, 0], ((1,), (1,))) * scale
        mx = jnp.max(logits, axis=1, keepdims=True)
        p = jnp.exp(logits - mx)
        attn = p / jnp.sum(p, axis=1, keepdims=True)
        ctx = _dot(attn, v_ref[0, 0], ((1,), (0,)))
        contrib = _dot(ctx, wo_ref[0], ((1,), (1,)))
        for i in range(NT):
            o_ref[0, pl.ds(ids[i], 1), :] += contrib[i:i + 1, :]
    return body


def kernel(queries, keys, values, Wq, bq, Wk, bk, Wv, bv, Wo, bo, index_sample):
    B, L, DM = queries.shape
    S = keys.shape[1]
    H = 16
    DK = DM // H
    SK = index_sample.shape[0]
    NT = max(1, min(5 * int(math.log(L)), L))
    NTP = 64
    SKP = ((SK + 7) // 8) * 8
    scale = 1.0 / math.sqrt(DK)

    bq2, bk2, bv2, bo2 = (x.reshape(1, DM) for x in (bq, bk, bv, bo))

    Q = _proj(queries, Wq, bq2, H, DK, 512)
    K = _proj(keys, Wk, bk2, H, DK, 512)
    V = _proj(values, Wv, bv2, H, DK, 512)

    idxp = jnp.pad(index_sample.reshape(1, SK), ((0, 0), (0, SKP - SK)),
                   mode="edge").astype(jnp.int32)

    Ks = pl.pallas_call(
        _make_gather_ks(H, SK, SKP),
        grid=(B,),
        in_specs=[
            pl.BlockSpec((1, H, S, DK), lambda b: (b, 0, 0, 0)),
            pl.BlockSpec(memory_space=pltpu.SMEM),
        ],
        out_specs=pl.BlockSpec((1, H, SKP, DK), lambda b: (b, 0, 0, 0)),
        out_shape=jax.ShapeDtypeStruct((B, H, SKP, DK), _F32),
    )(K, idxp)

    TL = 512
    M = pl.pallas_call(
        _make_score(H, DK, SK, TL),
        grid=(B, L // TL),
        in_specs=[
            pl.BlockSpec((1, H, TL, DK), lambda b, i: (b, 0, i, 0)),
            pl.BlockSpec((1, H, SKP, DK), lambda b, i: (b, 0, 0, 0)),
        ],
        out_specs=pl.BlockSpec((1, H, TL), lambda b, i: (b, 0, i)),
        out_shape=jax.ShapeDtypeStruct((B, H, L), _F32),
    )(Q, Ks)

    top_idx = pl.pallas_call(
        _make_topk(H, L, NT, NTP),
        grid=(B,),
        in_specs=[pl.BlockSpec((1, H, L), lambda b: (b, 0, 0))],
        out_specs=pl.BlockSpec((1, H, NTP), lambda b: (b, 0, 0)),
        out_shape=jax.ShapeDtypeStruct((B, H, NTP), jnp.int32),
    )(M)

    Wo3 = jnp.transpose(Wo.reshape(DM, H, DK), (1, 0, 2))

    out = pl.pallas_call(
        _make_attn(L, DM, DK, NT, scale),
        grid=(B, H),
        in_specs=[
            pl.BlockSpec((1, 1, L, DK), lambda b, h: (b, h, 0, 0)),
            pl.BlockSpec((1, 1, S, DK), lambda b, h: (b, h, 0, 0)),
            pl.BlockSpec((1, 1, S, DK), lambda b, h: (b, h, 0, 0)),
            pl.BlockSpec(memory_space=pltpu.SMEM),
            pl.BlockSpec((1, DM, DK), lambda b, h: (h, 0, 0)),
            pl.BlockSpec((1, DM), lambda b, h: (0, 0)),
        ],
        out_specs=pl.BlockSpec((1, L, DM), lambda b, h: (b, 0, 0)),
        out_shape=jax.ShapeDtypeStruct((B, L, DM), _F32),
    )(Q, K, V, top_idx, Wo3, bo2)

    return out


# speedup vs baseline: 1.9552x; 1.2234x over previous
import math

import jax
import jax.numpy as jnp
from jax.experimental import pallas as pl
from jax.experimental.pallas import tpu as pltpu

_PREC = jax.lax.Precision.DEFAULT
_F32 = jnp.float32


def _dot(a, b, dims):
    return jax.lax.dot_general(a, b, (dims, ((), ())), precision=_PREC,
                               preferred_element_type=_F32)


def _proj_body(x_ref, w_ref, b_ref, o_ref):
    o_ref[...] = _dot(x_ref[...], w_ref[...], ((1,), (1,))) + b_ref[...]


def _proj(x2d, W, b2d, tm):
    m, dm = x2d.shape
    return pl.pallas_call(
        _proj_body,
        grid=(m // tm,),
        in_specs=[
            pl.BlockSpec((tm, dm), lambda i: (i, 0)),
            pl.BlockSpec((dm, dm), lambda i: (0, 0)),
            pl.BlockSpec((1, dm), lambda i: (0, 0)),
        ],
        out_specs=pl.BlockSpec((tm, dm), lambda i: (i, 0)),
        out_shape=jax.ShapeDtypeStruct((m, dm), _F32),
    )(x2d, W, b2d)


def _make_gather_ks(SKP):
    def body(k_ref, idx_ref, o_ref):
        rows = [k_ref[0, pl.ds(idx_ref[0, i], 1), :] for i in range(SKP)]
        o_ref[0] = jnp.concatenate(rows, axis=0)
    return body


def _make_score(H, DK, SK):
    def body(q_ref, ks_ref, m_ref):
        q = q_ref[0]
        ks = ks_ref[0]
        rows = []
        for h in range(H):
            s = _dot(q[:, h * DK:(h + 1) * DK], ks[:SK, h * DK:(h + 1) * DK],
                     ((1,), (1,)))
            rows.append(jnp.max(s, axis=1) - jnp.sum(s, axis=1) * (1.0 / SK))
        m_ref[0] = jnp.stack(rows, axis=0)
    return body


def _make_topk(H, L, NT, NTP):
    def body(m_ref, o_ref):
        m = m_ref[0]
        iota = jax.lax.broadcasted_iota(jnp.int32, (H, L), 1)
        lane = jax.lax.broadcasted_iota(jnp.int32, (H, NTP), 1)
        acc = jnp.zeros((H, NTP), jnp.int32)
        for t in range(NT):
            mx = jnp.max(m, axis=1, keepdims=True)
            idx = jnp.min(jnp.where(m >= mx, iota, L), axis=1)
            acc = jnp.where(lane == t, idx[:, None], acc)
            m = jnp.where(iota == idx[:, None], -3.4e38, m)
        o_ref[0] = acc
    return body


def _make_attn(L, DM, DK, NT, scale):
    def body(q_ref, k_ref, v_ref, idx_ref, wo_ref, bo_ref, o_ref):
        b = pl.program_id(0)
        g = pl.program_id(1)

        @pl.when(g == 0)
        def _():
            o_ref[...] = jnp.broadcast_to(bo_ref[...][None], (1, L, DM))

        k2 = k_ref[0]
        v2 = v_ref[0]
        for sub in range(2):
            hh = 2 * g + sub
            cs = slice(sub * DK, (sub + 1) * DK)
            ids = [idx_ref[b, hh, i] for i in range(NT)]
            qr = jnp.concatenate(
                [q_ref[0, pl.ds(ids[i], 1), cs] for i in range(NT)], axis=0)
            logits = _dot(qr, k2[:, cs], ((1,), (1,))) * scale
            mx = jnp.max(logits, axis=1, keepdims=True)
            p = jnp.exp(logits - mx)
            attn = p / jnp.sum(p, axis=1, keepdims=True)
            ctx = _dot(attn, v2[:, cs], ((1,), (0,)))
            contrib = _dot(ctx, wo_ref[:, cs], ((1,), (1,)))
            for i in range(NT):
                o_ref[0, pl.ds(ids[i], 1), :] += contrib[i:i + 1, :]
    return body


def kernel(queries, keys, values, Wq, bq, Wk, bk, Wv, bv, Wo, bo, index_sample):
    B, L, DM = queries.shape
    S = keys.shape[1]
    H = 16
    DK = DM // H
    SK = index_sample.shape[0]
    NT = max(1, min(5 * int(math.log(L)), L))
    NTP = 64
    SKP = ((SK + 7) // 8) * 8
    scale = 1.0 / math.sqrt(DK)

    bq2, bk2, bv2, bo2 = (x.reshape(1, DM) for x in (bq, bk, bv, bo))

    Q = _proj(queries.reshape(B * L, DM), Wq, bq2, 512).reshape(B, L, DM)
    K = _proj(keys.reshape(B * S, DM), Wk, bk2, 512).reshape(B, S, DM)
    V = _proj(values.reshape(B * S, DM), Wv, bv2, 512).reshape(B, S, DM)

    idxp = jnp.pad(index_sample.reshape(1, SK), ((0, 0), (0, SKP - SK)),
                   mode="edge").astype(jnp.int32)

    Ks = pl.pallas_call(
        _make_gather_ks(SKP),
        grid=(B,),
        in_specs=[
            pl.BlockSpec((1, S, DM), lambda b: (b, 0, 0)),
            pl.BlockSpec(memory_space=pltpu.SMEM),
        ],
        out_specs=pl.BlockSpec((1, SKP, DM), lambda b: (b, 0, 0)),
        out_shape=jax.ShapeDtypeStruct((B, SKP, DM), _F32),
    )(K, idxp)

    TL = 512
    M = pl.pallas_call(
        _make_score(H, DK, SK),
        grid=(B, L // TL),
        in_specs=[
            pl.BlockSpec((1, TL, DM), lambda b, i: (b, i, 0)),
            pl.BlockSpec((1, SKP, DM), lambda b, i: (b, 0, 0)),
        ],
        out_specs=pl.BlockSpec((1, H, TL), lambda b, i: (b, 0, i)),
        out_shape=jax.ShapeDtypeStruct((B, H, L), _F32),
    )(Q, Ks)

    top_idx = pl.pallas_call(
        _make_topk(H, L, NT, NTP),
        grid=(B,),
        in_specs=[pl.BlockSpec((1, H, L), lambda b: (b, 0, 0))],
        out_specs=pl.BlockSpec((1, H, NTP), lambda b: (b, 0, 0)),
        out_shape=jax.ShapeDtypeStruct((B, H, NTP), jnp.int32),
    )(M)

    out = pl.pallas_call(
        _make_attn(L, DM, DK, NT, scale),
        grid=(B, H // 2),
        in_specs=[
            pl.BlockSpec((1, L, 2 * DK), lambda b, g: (b, 0, g)),
            pl.BlockSpec((1, S, 2 * DK), lambda b, g: (b, 0, g)),
            pl.BlockSpec((1, S, 2 * DK), lambda b, g: (b, 0, g)),
            pl.BlockSpec(memory_space=pltpu.SMEM),
            pl.BlockSpec((DM, 2 * DK), lambda b, g: (0, g)),
            pl.BlockSpec((1, DM), lambda b, g: (0, 0)),
        ],
        out_specs=pl.BlockSpec((1, L, DM), lambda b, g: (b, 0, 0)),
        out_shape=jax.ShapeDtypeStruct((B, L, DM), _F32),
    )(Q, K, V, top_idx, Wo, bo2)

    return out
